# baseline (speedup 1.0000x reference)
import jax
import jax.numpy as jnp
from jax import lax
from jax.experimental import pallas as pl
from jax.experimental.pallas import tpu as pltpu
from jax.experimental.pallas import tpu_sc as plsc

_SIZES = (500, 200, 2000, 1000, 300)
_OFF_RAW = (0, 500, 700, 2700, 3700)
_N_IN = 10000
_B, _T = 4, 2048
_TBLK = _T // 128
_ROW_STRIDE_B = _N_IN * _T // 128

_NC, _NS, _L = 2, 16, 16
_NW = _NC * _NS

_SIZES_P128 = (512, 256, 2048, 1024, 384)
_OFF_P128 = (0, 512, 768, 2816, 3840)
_TOTAL_IDS = 4224

_ALL_CHUNKS = tuple(
    (k, _OFF_P128[k] + j0, j0, min(128, _SIZES[k] - j0))
    for k in range(5)
    for j0 in range(0, _SIZES[k], 128)
)


def _body(y_ref, g_ref, o0, o1, o2, o3, o4,
          idx_v, rows0, rows1, ob0, ob1,
          sem_in0, sem_in1, sem_out0, sem_out1):
    outs = (o0, o1, o2, o3, o4)
    rows = (rows0, rows1)
    obs = (ob0, ob1)
    sems_in = (sem_in0, sem_in1)
    sems_out = (sem_out0, sem_out1)
    wid = lax.axis_index("s") * _NC + lax.axis_index("c")

    lane = lax.iota(jnp.int32, _L)
    cjg = [lane + 16 * jg for jg in range(8)]

    p0 = 2 * wid
    b = p0 // _TBLK
    th0 = p0 % _TBLK
    base0 = b * _ROW_STRIDE_B + th0 * 8

    pltpu.sync_copy(g_ref, idx_v)

    def add_const(c):
        @plsc.parallel_loop(0, _TOTAL_IDS // _L, 1, unroll=4)
        def _(i):
            idx_v[pl.ds(i * _L, _L)] = idx_v[pl.ds(i * _L, _L)] + c

    add_const(base0)

    nchunks = len(_ALL_CHUNKS)

    def out_wait(slot, t0):
        pltpu.make_async_copy(
            obs[slot],
            outs[2].at[b, pl.ds(t0, 128), pl.ds(0, 128)],
            sems_out[slot],
        ).wait()

    def do_pair(pi, carry):
        add_const(jnp.where(pi == 0, 0, 8))
        t0 = (th0 + pi) * 128
        pending = [False, False]
        first_use = [True, True]

        def start_in(ci):
            _, ioff, _, _ = _ALL_CHUNKS[ci]
            return pltpu.async_copy(
                y_ref.at[idx_v.at[pl.ds(ioff, 128)]],
                rows[ci % 2], sems_in[ci % 2],
            )

        in_flight = {0: start_in(0)}
        for ci, (k, ioff, j0, w) in enumerate(_ALL_CHUNKS):
            slot = ci % 2
            if ci + 1 < nchunks:
                in_flight[ci + 1] = start_in(ci + 1)
            in_flight.pop(ci).wait()

            rbuf = rows[slot]
            nfull = w // _L
            tail = w % _L

            def tr_into(obuf, rbuf=rbuf, nfull=nfull, tail=tail):
                @plsc.parallel_loop(0, 128, 1, unroll=4)
                def _(t):
                    tsp = jnp.full((_L,), 0, jnp.int32) + t
                    for jg in range(nfull):
                        vals = plsc.load_gather(rbuf, [cjg[jg], tsp])
                        obuf[t, pl.ds(jg * _L, _L)] = vals
                    if tail:
                        vals = plsc.load_gather(rbuf, [cjg[nfull], tsp])
                        plsc.store_scatter(
                            obuf, [tsp, lane + nfull * _L], vals,
                            mask=lane < tail,
                        )

            dst = outs[k].at[b, pl.ds(t0, 128), pl.ds(j0, w)]
            if w == 128:
                if pending[slot]:
                    out_wait(slot, t0)
                elif first_use[slot] and slot == 1:
                    @pl.when(pi > 0)
                    def _():
                        out_wait(1, t0)
                first_use[slot] = False
                obuf = obs[slot]
                tr_into(obuf)
                pltpu.async_copy(obuf, dst, sems_out[slot])
                pending[slot] = True
            else:
                def tail_chunk(obuf, dst=dst):
                    tr_into(obuf)
                    pltpu.async_copy(obuf, dst, sems_out[0]).wait()

                if pending[0]:
                    out_wait(0, t0)
                    pending[0] = False
                pl.run_scoped(
                    tail_chunk, pltpu.VMEM((128, w), jnp.float32)
                )
        assert pending == [False, True], pending
        return carry

    lax.fori_loop(0, 2, do_pair, 0)

    pltpu.make_async_copy(
        obs[1],
        outs[2].at[b, pl.ds(th0 * 128, 128), pl.ds(0, 128)],
        sems_out[1],
    ).wait()


@jax.jit
def kernel(x, cat_ids):
    b, t, n = x.shape

    y2d = (
        x.transpose(0, 2, 1)
        .reshape(b, n // 8, 8, t // 128, 128)
        .transpose(0, 1, 3, 2, 4)
        .reshape(b * n * t // 128, 128)
    )

    segs = []
    for k, (s, sp) in enumerate(zip(_SIZES, _SIZES_P128)):
        seg = lax.dynamic_slice(cat_ids, (_OFF_RAW[k],), (s,))
        segs.append(jnp.pad(seg, (0, sp - s)))
    ids_all = jnp.concatenate(segs)
    g = (ids_all // 8) * 128 + (ids_all % 8)

    mesh = plsc.VectorSubcoreMesh(
        core_axis_name="c", subcore_axis_name="s", num_cores=_NC, num_subcores=_NS
    )
    out_type = tuple(
        jax.ShapeDtypeStruct((b, t, s), jnp.float32) for s in _SIZES
    )
    fn = pl.kernel(
        _body,
        out_type=out_type,
        mesh=mesh,
        compiler_params=pltpu.CompilerParams(needs_layout_passes=False),
        scratch_types=[
            pltpu.VMEM((_TOTAL_IDS,), jnp.int32),
            pltpu.VMEM((128, 128), jnp.float32),
            pltpu.VMEM((128, 128), jnp.float32),
            pltpu.VMEM((128, 128), jnp.float32),
            pltpu.VMEM((128, 128), jnp.float32),
            pltpu.SemaphoreType.DMA,
            pltpu.SemaphoreType.DMA,
            pltpu.SemaphoreType.DMA,
            pltpu.SemaphoreType.DMA,
        ],
    )
    return fn(y2d, g)


# speedup vs baseline: 1.0072x; 1.0072x over previous
import jax
import jax.numpy as jnp
from jax import lax
from jax.experimental import pallas as pl
from jax.experimental.pallas import tpu as pltpu
from jax.experimental.pallas import tpu_sc as plsc

_SIZES = (500, 200, 2000, 1000, 300)
_OFF_RAW = (0, 500, 700, 2700, 3700)
_N_IN = 10000
_B, _T = 4, 2048
_TBLK = _T // 128
_ROW_STRIDE_B = _N_IN * _T // 128

_NC, _NS, _L = 2, 16, 16
_NW = _NC * _NS

_SIZES_P128 = (512, 256, 2048, 1024, 384)
_OFF_P128 = (0, 512, 768, 2816, 3840)
_TOTAL_IDS = 4224

_ALL_CHUNKS = tuple(
    (k, _OFF_P128[k] + j0, j0, min(128, _SIZES[k] - j0))
    for k in range(5)
    for j0 in range(0, _SIZES[k], 128)
)


def _body(y_ref, g_ref, o0, o1, o2, o3, o4,
          idx_v, rows0, rows1, rows2, ob0, ob1,
          sem_in0, sem_in1, sem_in2, sem_out0, sem_out1):
    outs = (o0, o1, o2, o3, o4)
    rows = (rows0, rows1, rows2)
    obs = (ob0, ob1)
    sems_in = (sem_in0, sem_in1, sem_in2)
    sems_out = (sem_out0, sem_out1)
    wid = lax.axis_index("s") * _NC + lax.axis_index("c")

    lane = lax.iota(jnp.int32, _L)
    cjg = [lane + 16 * jg for jg in range(8)]

    p0 = 2 * wid
    b = p0 // _TBLK
    th0 = p0 % _TBLK
    base0 = b * _ROW_STRIDE_B + th0 * 8

    pltpu.sync_copy(g_ref, idx_v)

    def add_const(c):
        @plsc.parallel_loop(0, _TOTAL_IDS // _L, 1, unroll=4)
        def _(i):
            idx_v[pl.ds(i * _L, _L)] = idx_v[pl.ds(i * _L, _L)] + c

    add_const(base0)

    nchunks = len(_ALL_CHUNKS)

    def out_wait(slot, t0):
        pltpu.make_async_copy(
            obs[slot],
            outs[2].at[b, pl.ds(t0, 128), pl.ds(0, 128)],
            sems_out[slot],
        ).wait()

    def do_pair(pi, carry):
        add_const(jnp.where(pi == 0, 0, 8))
        t0 = (th0 + pi) * 128
        pending = [False, False]
        first_use = [True, True]

        def start_in(ci):
            _, ioff, _, _ = _ALL_CHUNKS[ci]
            return pltpu.async_copy(
                y_ref.at[idx_v.at[pl.ds(ioff, 128)]],
                rows[ci % 3], sems_in[ci % 3],
            )

        in_flight = {0: start_in(0), 1: start_in(1)}
        for ci, (k, ioff, j0, w) in enumerate(_ALL_CHUNKS):
            slot = ci % 2
            if ci + 2 < nchunks:
                in_flight[ci + 2] = start_in(ci + 2)
            in_flight.pop(ci).wait()

            rbuf = rows[ci % 3]
            nfull = w // _L
            tail = w % _L

            def tr_into(obuf, rbuf=rbuf, nfull=nfull, tail=tail):
                @plsc.parallel_loop(0, 128, 1, unroll=4)
                def _(t):
                    tsp = jnp.full((_L,), 0, jnp.int32) + t
                    for jg in range(nfull):
                        vals = plsc.load_gather(rbuf, [cjg[jg], tsp])
                        obuf[t, pl.ds(jg * _L, _L)] = vals
                    if tail:
                        vals = plsc.load_gather(rbuf, [cjg[nfull], tsp])
                        plsc.store_scatter(
                            obuf, [tsp, lane + nfull * _L], vals,
                            mask=lane < tail,
                        )

            dst = outs[k].at[b, pl.ds(t0, 128), pl.ds(j0, w)]
            if w == 128:
                if pending[slot]:
                    out_wait(slot, t0)
                elif first_use[slot] and slot == 1:
                    @pl.when(pi > 0)
                    def _():
                        out_wait(1, t0)
                first_use[slot] = False
                obuf = obs[slot]
                tr_into(obuf)
                pltpu.async_copy(obuf, dst, sems_out[slot])
                pending[slot] = True
            else:
                def tail_chunk(obuf, dst=dst):
                    tr_into(obuf)
                    pltpu.async_copy(obuf, dst, sems_out[0]).wait()

                if pending[0]:
                    out_wait(0, t0)
                    pending[0] = False
                pl.run_scoped(
                    tail_chunk, pltpu.VMEM((128, w), jnp.float32)
                )
        assert pending == [False, True], pending
        return carry

    lax.fori_loop(0, 2, do_pair, 0)

    pltpu.make_async_copy(
        obs[1],
        outs[2].at[b, pl.ds(th0 * 128, 128), pl.ds(0, 128)],
        sems_out[1],
    ).wait()


@jax.jit
def kernel(x, cat_ids):
    b, t, n = x.shape

    y2d = (
        x.transpose(0, 2, 1)
        .reshape(b, n // 8, 8, t // 128, 128)
        .transpose(0, 1, 3, 2, 4)
        .reshape(b * n * t // 128, 128)
    )

    segs = []
    for k, (s, sp) in enumerate(zip(_SIZES, _SIZES_P128)):
        seg = lax.dynamic_slice(cat_ids, (_OFF_RAW[k],), (s,))
        segs.append(jnp.pad(seg, (0, sp - s)))
    ids_all = jnp.concatenate(segs)
    g = (ids_all // 8) * 128 + (ids_all % 8)

    mesh = plsc.VectorSubcoreMesh(
        core_axis_name="c", subcore_axis_name="s", num_cores=_NC, num_subcores=_NS
    )
    out_type = tuple(
        jax.ShapeDtypeStruct((b, t, s), jnp.float32) for s in _SIZES
    )
    fn = pl.kernel(
        _body,
        out_type=out_type,
        mesh=mesh,
        compiler_params=pltpu.CompilerParams(needs_layout_passes=False),
        scratch_types=[
            pltpu.VMEM((_TOTAL_IDS,), jnp.int32),
            pltpu.VMEM((128, 128), jnp.float32),
            pltpu.VMEM((128, 128), jnp.float32),
            pltpu.VMEM((128, 128), jnp.float32),
            pltpu.VMEM((128, 128), jnp.float32),
            pltpu.VMEM((128, 128), jnp.float32),
            pltpu.SemaphoreType.DMA,
            pltpu.SemaphoreType.DMA,
            pltpu.SemaphoreType.DMA,
            pltpu.SemaphoreType.DMA,
            pltpu.SemaphoreType.DMA,
        ],
    )
    return fn(y2d, g)
